# baseline (speedup 1.0000x reference)
import functools

import jax
import jax.numpy as jnp
from jax import lax
from jax.experimental import pallas as pl
from jax.experimental.pallas import tpu as pltpu
from jax.experimental.pallas import tpu_sc as plsc

CHUNK = 128
GROUP = 128
LANES = 16


@functools.cache
def _make_gather2(B, D):
    D2 = 2 * D
    info = plsc.get_sparse_core_info()
    nw = info.num_cores * info.num_subcores
    rows_per_worker = B // nw
    n_pairs = rows_per_worker // (2 * GROUP)
    gpb = GROUP // CHUNK
    mesh = plsc.VectorSubcoreMesh(core_axis_name="c", subcore_axis_name="s")

    @functools.partial(
        pl.kernel,
        mesh=mesh,
        compiler_params=pltpu.CompilerParams(use_tc_tiling_on_sc=True),
        out_type=[
            jax.ShapeDtypeStruct((B, D), jnp.float32),
            jax.ShapeDtypeStruct((B, D), jnp.float32),
        ],
        scratch_types=[
            pltpu.VMEM((GROUP,), jnp.int32),
            pltpu.VMEM((GROUP,), jnp.int32),
            pltpu.VMEM((GROUP,), jnp.int32),
            pltpu.VMEM((GROUP,), jnp.int32),
            pltpu.VMEM((GROUP, D2), jnp.float32),
            pltpu.VMEM((GROUP, D2), jnp.float32),
            pltpu.VMEM((GROUP, D), jnp.float32),
            pltpu.VMEM((GROUP, D), jnp.float32),
            pltpu.SemaphoreType.DMA,
            pltpu.SemaphoreType.DMA,
            pltpu.SemaphoreType.DMA,
            pltpu.SemaphoreType.DMA,
        ],
    )
    def gather2(qi_hbm, vi_hbm, qt_hbm, kt_hbm, qo_hbm, vo_hbm,
                idx0, idx1, half0, half1, rows0, rows1, comp0, comp1,
                sg0, sg1, sw0, sw1):
        wid = lax.axis_index("s") * info.num_cores + lax.axis_index("c")
        base = wid * rows_per_worker

        def run_table(idx_hbm, tab_hbm, out_hbm):
            def fill(idx_v, half_v, rows_v, gsem, off):
                pltpu.sync_copy(idx_hbm.at[pl.ds(off, GROUP)], idx_v)
                for k in range(GROUP // LANES):
                    sl = pl.ds(k * LANES, LANES)
                    half_v[sl] = lax.shift_right_logical(idx_v[sl], 1)
                return [
                    pltpu.async_copy(
                        tab_hbm.at[half_v.at[pl.ds(j * CHUNK, CHUNK)]],
                        rows_v.at[pl.ds(j * CHUNK, CHUNK)],
                        gsem,
                    )
                    for j in range(gpb)
                ]

            def compact(idx_v, rows_v, comp_v):
                def crow(jb, carry):
                    j0 = jb * LANES
                    starts = (idx_v[pl.ds(j0, LANES)] & 1) * D
                    for l in range(LANES):
                        start = starts[l]
                        for k in range(D // LANES):
                            comp_v[j0 + l, pl.ds(k * LANES, LANES)] = (
                                rows_v[j0 + l, pl.ds(start + k * LANES, LANES)])
                    return carry

                lax.fori_loop(0, GROUP // LANES, crow, 0)

            def step(i, carry):
                off0 = base + (2 * i) * GROUP
                off1 = off0 + GROUP

                @pl.when(i > 0)
                def _():
                    pltpu.make_async_copy(
                        comp0, out_hbm.at[pl.ds(off0, GROUP)], sw0).wait()

                cps0 = fill(idx0, half0, rows0, sg0, off0)

                @pl.when(i > 0)
                def _():
                    pltpu.make_async_copy(
                        comp1, out_hbm.at[pl.ds(off1, GROUP)], sw1).wait()

                cps1 = fill(idx1, half1, rows1, sg1, off1)

                for c in cps0:
                    c.wait()
                compact(idx0, rows0, comp0)
                pltpu.async_copy(comp0, out_hbm.at[pl.ds(off0, GROUP)], sw0)
                for c in cps1:
                    c.wait()
                compact(idx1, rows1, comp1)
                pltpu.async_copy(comp1, out_hbm.at[pl.ds(off1, GROUP)], sw1)
                return carry

            lax.fori_loop(0, n_pairs, step, 0)
            pltpu.make_async_copy(comp0, out_hbm.at[pl.ds(base, GROUP)], sw0).wait()
            pltpu.make_async_copy(comp1, out_hbm.at[pl.ds(base, GROUP)], sw1).wait()

        run_table(qi_hbm, qt_hbm, qo_hbm)
        run_table(vi_hbm, kt_hbm, vo_hbm)

    return gather2


def kernel(queries, values, query_table, key_table):
    batch, hist = queries.shape
    n_rows, depth = query_table.shape
    flat = batch * hist
    qi = queries.reshape(flat).astype(jnp.int32)
    vi = values.reshape(flat).astype(jnp.int32)
    qt2 = query_table.reshape(n_rows // 2, 2 * depth)
    kt2 = key_table.reshape(n_rows // 2, 2 * depth)
    q_out, v_out = _make_gather2(flat, depth)(qi, vi, qt2, kt2)
    return q_out.reshape(batch, hist, depth), v_out.reshape(batch, hist, depth)


# speedup vs baseline: 1.0184x; 1.0184x over previous
import functools

import jax
import jax.numpy as jnp
from jax import lax
from jax.experimental import pallas as pl
from jax.experimental.pallas import tpu as pltpu
from jax.experimental.pallas import tpu_sc as plsc

LANES = 16
OCT = 8


@functools.cache
def _make_gather2(batch, hist, depth):
    d2 = 2 * depth
    info = plsc.get_sparse_core_info()
    nw = info.num_cores * info.num_subcores
    entries_per_worker = batch // nw
    n_oct = entries_per_worker // OCT
    blocks = list(range(0, hist - LANES + 1, LANES))
    if blocks[-1] + LANES < hist:
        blocks.append(hist - LANES)
    mesh = plsc.VectorSubcoreMesh(core_axis_name="c", subcore_axis_name="s")

    @functools.partial(
        pl.kernel,
        mesh=mesh,
        compiler_params=pltpu.CompilerParams(use_tc_tiling_on_sc=True),
        out_type=[
            jax.ShapeDtypeStruct((batch, hist, depth), jnp.float32),
            jax.ShapeDtypeStruct((batch, hist, depth), jnp.float32),
        ],
        scratch_types=[
            pltpu.VMEM((OCT, hist), jnp.int32),
            pltpu.VMEM((OCT, hist), jnp.int32),
            pltpu.VMEM((hist, d2), jnp.float32),
            pltpu.VMEM((hist, d2), jnp.float32),
            pltpu.VMEM((hist, depth), jnp.float32),
            pltpu.VMEM((hist, depth), jnp.float32),
            pltpu.SemaphoreType.DMA,
            pltpu.SemaphoreType.DMA,
            pltpu.SemaphoreType.DMA,
            pltpu.SemaphoreType.DMA,
        ],
    )
    def gather2(qi_hbm, vi_hbm, qt_hbm, kt_hbm, qo_hbm, vo_hbm,
                idx_v, half_v, rows0, rows1, comp0, comp1,
                sg0, sg1, sw0, sw1):
        wid = lax.axis_index("s") * info.num_cores + lax.axis_index("c")
        ebase = wid * entries_per_worker
        rows = (rows0, rows1)
        comp = (comp0, comp1)
        sg = (sg0, sg1)
        sw = (sw0, sw1)

        def run_table(idx_hbm, tab_hbm, out_hbm):
            def fire(e, slot):
                for st in blocks:
                    sl = pl.ds(st, LANES)
                    half_v[e, sl] = lax.shift_right_logical(idx_v[e, sl], 1)
                return [
                    pltpu.async_copy(
                        tab_hbm.at[half_v.at[e, pl.ds(0, 128)]],
                        rows[slot].at[pl.ds(0, 128)],
                        sg[slot],
                    ),
                    pltpu.async_copy(
                        tab_hbm.at[half_v.at[e, pl.ds(128, hist - 128)]],
                        rows[slot].at[pl.ds(128, hist - 128)],
                        sg[slot],
                    ),
                ]

            def compact(e, slot):
                rows_v, comp_v = rows[slot], comp[slot]

                def cblock(kb, carry):
                    st = kb * LANES
                    starts = (idx_v[e, pl.ds(st, LANES)] & 1) * depth
                    for l in range(LANES):
                        s0 = starts[l]
                        for k in range(depth // LANES):
                            comp_v[st + l, pl.ds(k * LANES, LANES)] = (
                                rows_v[st + l, pl.ds(s0 + k * LANES, LANES)])
                    return carry

                lax.fori_loop(0, hist // LANES, cblock, 0)
                st = hist - LANES
                starts = (idx_v[e, pl.ds(st, LANES)] & 1) * depth
                for l in range(LANES):
                    s0 = starts[l]
                    for k in range(depth // LANES):
                        comp_v[st + l, pl.ds(k * LANES, LANES)] = (
                            rows_v[st + l, pl.ds(s0 + k * LANES, LANES)])

            def octet(o, carry):
                b0 = ebase + o * OCT
                pltpu.sync_copy(idx_hbm.at[pl.ds(b0, OCT)], idx_v)
                cps = {0: fire(0, 0)}
                for e in range(OCT):
                    slot = e % 2
                    if e + 1 < OCT:
                        cps[e + 1] = fire(e + 1, 1 - slot)
                    for c in cps.pop(e):
                        c.wait()
                    @pl.when(jnp.logical_or(o > 0, e >= 2))
                    def _():
                        pltpu.make_async_copy(
                            comp[slot], out_hbm.at[b0 + e], sw[slot]).wait()

                    compact(e, slot)
                    pltpu.async_copy(comp[slot], out_hbm.at[b0 + e], sw[slot])
                return carry

            lax.fori_loop(0, n_oct, octet, 0)
            pltpu.make_async_copy(comp0, out_hbm.at[ebase], sw0).wait()
            pltpu.make_async_copy(comp1, out_hbm.at[ebase], sw1).wait()

        run_table(qi_hbm, qt_hbm, qo_hbm)
        run_table(vi_hbm, kt_hbm, vo_hbm)

    return gather2


def kernel(queries, values, query_table, key_table):
    batch, hist = queries.shape
    n_rows, depth = query_table.shape
    qi = queries.astype(jnp.int32)
    vi = values.astype(jnp.int32)
    qt2 = query_table.reshape(n_rows // 2, 2 * depth)
    kt2 = key_table.reshape(n_rows // 2, 2 * depth)
    q_out, v_out = _make_gather2(batch, hist, depth)(qi, vi, qt2, kt2)
    return q_out, v_out
